# baseline (speedup 1.0000x reference)
import functools

import jax
import jax.numpy as jnp
from jax import lax
from jax.experimental import pallas as pl
from jax.experimental.pallas import tpu as pltpu
from jax.experimental.pallas import tpu_sc as plsc

EMBED = 64
PADE = 128
NUM_CORES = 2
NUM_SUBCORES = 16
NW = NUM_CORES * NUM_SUBCORES
CHUNK = 800
DEPAD_K = 4000
EPS = 1e-5


def _mesh():
    return plsc.VectorSubcoreMesh(
        core_axis_name="c", subcore_axis_name="s",
        num_cores=NUM_CORES, num_subcores=NUM_SUBCORES)


def _depad_body(x_ref, o_ref):
    h = pl.program_id(1)

    @pl.when(h == 0)
    def _():
        o_ref[:, :EMBED] = x_ref[...]

    @pl.when(h == 1)
    def _():
        o_ref[:, EMBED:] = x_ref[...]


def _tc_depad(table):
    v_rows = table.shape[0]
    half_blocks = (v_rows // 2) // DEPAD_K
    return pl.pallas_call(
        _depad_body,
        grid=(half_blocks, 2),
        in_specs=[
            pl.BlockSpec((DEPAD_K, EMBED),
                         lambda i, h: (h * half_blocks + i, 0))],
        out_specs=pl.BlockSpec((DEPAD_K, PADE), lambda i, h: (i, 0)),
        out_shape=jax.ShapeDtypeStruct((v_rows // 2, PADE), jnp.float32),
    )(table)


def _sc_gather(ids_flat, table_lin):
    n_ids = ids_flat.shape[0]
    ids_per_w = n_ids // NW
    n_chunks = ids_per_w // CHUNK

    @functools.partial(
        pl.kernel,
        out_type=jax.ShapeDtypeStruct((n_ids, PADE), jnp.float32),
        mesh=_mesh(),
        scratch_types=[
            pltpu.VMEM((ids_per_w,), jnp.int32),
            pltpu.VMEM((2, CHUNK, EMBED), jnp.float32),
            pltpu.SemaphoreType.DMA,
            pltpu.SemaphoreType.DMA,
            pltpu.SemaphoreType.DMA,
            pltpu.SemaphoreType.DMA,
        ],
        compiler_params=pltpu.CompilerParams(use_tc_tiling_on_sc=False),
    )
    def k(ids_hbm, table_hbm, out_hbm, idx_v, rows_v, g0, g1, w0, w1):
        wid = lax.axis_index("s") * NUM_CORES + lax.axis_index("c")
        base = wid * ids_per_w
        pltpu.sync_copy(ids_hbm.at[pl.ds(base, ids_per_w)], idx_v)
        gsems = (g0, g1)
        wsems = (w0, w1)

        def gather(c):
            b = c % 2
            return pltpu.async_copy(
                table_hbm.at[idx_v.at[pl.ds(c * CHUNK, CHUNK)]],
                rows_v.at[b], gsems[b])

        def write(c):
            b = c % 2
            return pltpu.async_copy(
                rows_v.at[b],
                out_hbm.at[pl.ds(base + c * CHUNK, CHUNK), pl.ds(0, EMBED)],
                wsems[b])

        pending_w = [None, None]
        gather(0)
        for c in range(n_chunks):
            b = c % 2
            if c + 1 < n_chunks:
                if pending_w[1 - b] is not None:
                    pending_w[1 - b].wait()
                    pending_w[1 - b] = None
                gather(c + 1)
            pltpu.make_async_copy(
                table_hbm.at[idx_v.at[pl.ds(c * CHUNK, CHUNK)]],
                rows_v.at[b], gsems[b]).wait()
            pending_w[b] = write(c)
        for p in pending_w:
            if p is not None:
                p.wait()

    return k(ids_flat, table_lin)


def _ln_body(x_ref, pos_ref, gamma_ref, beta_ref, o_ref):
    x = x_ref[:, :EMBED] + pos_ref[...]
    mean = jnp.mean(x, axis=-1, keepdims=True)
    cent = x - mean
    var = jnp.mean(cent * cent, axis=-1, keepdims=True)
    xhat = cent * lax.rsqrt(var + EPS)
    y = xhat * gamma_ref[...] + beta_ref[...]
    nb = o_ref.shape[0]
    seq = o_ref.shape[2]
    o_ref[...] = jnp.swapaxes(y.reshape(nb, seq, EMBED), 1, 2)


def _tc_ln(x, pos_tiled, gamma2d, beta2d, batch, seq):
    n, _ = x.shape
    rb = pos_tiled.shape[0]
    sb = rb // seq
    return pl.pallas_call(
        _ln_body,
        grid=(n // rb,),
        in_specs=[
            pl.BlockSpec((rb, PADE), lambda i: (i, 0)),
            pl.BlockSpec((rb, EMBED), lambda i: (0, 0)),
            pl.BlockSpec((1, EMBED), lambda i: (0, 0)),
            pl.BlockSpec((1, EMBED), lambda i: (0, 0)),
        ],
        out_specs=pl.BlockSpec((sb, EMBED, seq), lambda i: (i, 0, 0)),
        out_shape=jax.ShapeDtypeStruct((batch, EMBED, seq), jnp.float32),
    )(x, pos_tiled, gamma2d, beta2d)


def kernel(input_ids, word_table, pos_table, gamma, beta):
    B, L = input_ids.shape
    packed = _tc_depad(word_table)
    table_lin = packed.reshape(word_table.shape[0], EMBED)
    half = word_table.shape[0] // 2
    ids = input_ids.reshape(-1)
    ids_folded = 2 * (ids % half) + ids // half
    gathered = _sc_gather(ids_folded, table_lin)
    pos_tiled = jnp.tile(pos_table[:L], (8, 1))
    out_t = _tc_ln(gathered, pos_tiled, gamma.reshape(1, EMBED),
                   beta.reshape(1, EMBED), B, L)
    return out_t.transpose(0, 2, 1)


# speedup vs baseline: 1.3136x; 1.3136x over previous
import functools

import jax
import jax.numpy as jnp
from jax import lax
from jax.experimental import pallas as pl
from jax.experimental.pallas import tpu as pltpu
from jax.experimental.pallas import tpu_sc as plsc

EMBED = 64
PADE = 128
NUM_CORES = 2
NUM_SUBCORES = 16
NW = NUM_CORES * NUM_SUBCORES
CHUNK = 800
DEPAD_K = 4000
EPS = 1e-5


def _mesh():
    return plsc.VectorSubcoreMesh(
        core_axis_name="c", subcore_axis_name="s",
        num_cores=NUM_CORES, num_subcores=NUM_SUBCORES)


def _fold_pack(table):
    half = table.shape[0] // 2
    return jnp.concatenate([table[:half], table[half:]], axis=1).reshape(-1)


def _sc_gather(ids_flat, table_lin):
    n_ids = ids_flat.shape[0]
    ids_per_w = n_ids // NW
    n_chunks = ids_per_w // CHUNK

    @functools.partial(
        pl.kernel,
        out_type=jax.ShapeDtypeStruct((n_ids, PADE), jnp.float32),
        mesh=_mesh(),
        scratch_types=[
            pltpu.VMEM((ids_per_w,), jnp.int32),
            pltpu.VMEM((2, CHUNK, EMBED), jnp.float32),
            pltpu.SemaphoreType.DMA,
            pltpu.SemaphoreType.DMA,
            pltpu.SemaphoreType.DMA,
            pltpu.SemaphoreType.DMA,
        ],
        compiler_params=pltpu.CompilerParams(use_tc_tiling_on_sc=False),
    )
    def k(ids_hbm, table_hbm, out_hbm, idx_v, rows_v, g0, g1, w0, w1):
        wid = lax.axis_index("s") * NUM_CORES + lax.axis_index("c")
        base = wid * ids_per_w
        pltpu.sync_copy(ids_hbm.at[pl.ds(base, ids_per_w)], idx_v)
        gsems = (g0, g1)
        wsems = (w0, w1)

        def gather(c):
            b = c % 2
            return pltpu.async_copy(
                table_hbm.at[idx_v.at[pl.ds(c * CHUNK, CHUNK)]],
                rows_v.at[b], gsems[b])

        def write(c):
            b = c % 2
            return pltpu.async_copy(
                rows_v.at[b],
                out_hbm.at[pl.ds(base + c * CHUNK, CHUNK), pl.ds(0, EMBED)],
                wsems[b])

        pending_w = [None, None]
        gather(0)
        for c in range(n_chunks):
            b = c % 2
            if c + 1 < n_chunks:
                if pending_w[1 - b] is not None:
                    pending_w[1 - b].wait()
                    pending_w[1 - b] = None
                gather(c + 1)
            pltpu.make_async_copy(
                table_hbm.at[idx_v.at[pl.ds(c * CHUNK, CHUNK)]],
                rows_v.at[b], gsems[b]).wait()
            pending_w[b] = write(c)
        for p in pending_w:
            if p is not None:
                p.wait()

    return k(ids_flat, table_lin)


def _ln_body(x_ref, pos_ref, gamma_ref, beta_ref, o_ref):
    x = x_ref[:, :EMBED] + pos_ref[...]
    mean = jnp.mean(x, axis=-1, keepdims=True)
    cent = x - mean
    var = jnp.mean(cent * cent, axis=-1, keepdims=True)
    xhat = cent * lax.rsqrt(var + EPS)
    o_ref[...] = xhat * gamma_ref[...] + beta_ref[...]


def _tc_ln(x, pos_tiled, gamma2d, beta2d):
    n, _ = x.shape
    rb = pos_tiled.shape[0]
    return pl.pallas_call(
        _ln_body,
        grid=(n // rb,),
        in_specs=[
            pl.BlockSpec((rb, PADE), lambda i: (i, 0)),
            pl.BlockSpec((rb, EMBED), lambda i: (0, 0)),
            pl.BlockSpec((1, EMBED), lambda i: (0, 0)),
            pl.BlockSpec((1, EMBED), lambda i: (0, 0)),
        ],
        out_specs=pl.BlockSpec((rb, EMBED), lambda i: (i, 0)),
        out_shape=jax.ShapeDtypeStruct((n, EMBED), jnp.float32),
    )(x, pos_tiled, gamma2d, beta2d)


def kernel(input_ids, word_table, pos_table, gamma, beta):
    B, L = input_ids.shape
    packed = _fold_pack(word_table)
    table_lin = packed.reshape(word_table.shape[0], EMBED)
    half = word_table.shape[0] // 2
    ids = input_ids.reshape(-1)
    ids_folded = 2 * (ids % half) + ids // half
    gathered = _sc_gather(ids_folded, table_lin)
    pos_tiled = jnp.tile(pos_table[:L], (8, 1))
    out = _tc_ln(gathered, pos_tiled, gamma.reshape(1, EMBED),
                 beta.reshape(1, EMBED))
    return out.reshape(B, L, EMBED)
